# Initial kernel scaffold; baseline (speedup 1.0000x reference)
from functools import partial

import jax
import jax.numpy as jnp
from jax.experimental import pallas as pl

N = 10000
E = 160000
IN = 131
HID = 256
HEADS = 4
OUT = 6
D = HID * HEADS
BN = 1000
KPAD = 256

_NEG = -3.0e38


def _mm_first_body(x_ref, w_ref, aab_ref, h_ref, asad_ref, gmax_ref):
    i = pl.program_id(0)
    x = x_ref[...]
    h = jnp.dot(x, w_ref[...], preferred_element_type=jnp.float32)
    h_ref[...] = h
    asad = jnp.dot(h, aab_ref[...], preferred_element_type=jnp.float32)
    asad_ref[...] = asad
    part = jnp.max(asad, axis=0, keepdims=True)

    @pl.when(i == 0)
    def _():
        gmax_ref[...] = jnp.full((1, 8), _NEG, jnp.float32)

    gmax_ref[...] = jnp.maximum(gmax_ref[...], part)


def _mm_mid_body(acc_ref, den_ref, res_ref, b_ref, w_ref, aab_ref, rep_ref,
                 g_ref, h_ref, asad_ref, gmax_ref, *, relu):
    i = pl.program_id(0)
    den = jnp.dot(den_ref[...], rep_ref[...],
                  preferred_element_type=jnp.float32)
    g = acc_ref[...] / (den + 1e-16) + b_ref[...]
    if res_ref is not None:
        g = g + res_ref[...]
    if relu:
        g = jnp.maximum(g, 0.0)
    g_ref[...] = g
    h = jnp.dot(g, w_ref[...], preferred_element_type=jnp.float32)
    h_ref[...] = h
    asad = jnp.dot(h, aab_ref[...], preferred_element_type=jnp.float32)
    asad_ref[...] = asad
    part = jnp.max(asad, axis=0, keepdims=True)

    @pl.when(i == 0)
    def _():
        gmax_ref[...] = jnp.full((1, 8), _NEG, jnp.float32)

    gmax_ref[...] = jnp.maximum(gmax_ref[...], part)


def _mm_final_body(acc_ref, den_ref, res_ref, b_ref, wf_ref, bf_ref, rep_ref,
                   probs_ref, logits_ref):
    den = jnp.dot(den_ref[...], rep_ref[...],
                  preferred_element_type=jnp.float32)
    g = acc_ref[...] / (den + 1e-16) + b_ref[...] + res_ref[...]
    g = jnp.maximum(g, 0.0)
    logits = jnp.dot(g, wf_ref[...], preferred_element_type=jnp.float32)
    logits = logits + bf_ref[...]
    logits_ref[...] = logits
    z = logits - jnp.max(logits, axis=1, keepdims=True)
    ez = jnp.exp(z)
    probs_ref[...] = ez / jnp.sum(ez, axis=1, keepdims=True)


def _row_spec(cols):
    return pl.BlockSpec((BN, cols), lambda i: (i, 0))


def _full_spec(r, c):
    return pl.BlockSpec((r, c), lambda i: (0, 0))


def _mm_first(x_pad, w_pad, aab):
    grid = N // BN
    return pl.pallas_call(
        _mm_first_body,
        grid=(grid,),
        in_specs=[_row_spec(KPAD), _full_spec(KPAD, D), _full_spec(D, 8)],
        out_specs=[_row_spec(D), _row_spec(8), _full_spec(1, 8)],
        out_shape=[
            jax.ShapeDtypeStruct((N, D), jnp.float32),
            jax.ShapeDtypeStruct((N, 8), jnp.float32),
            jax.ShapeDtypeStruct((1, 8), jnp.float32),
        ],
    )(x_pad, w_pad, aab)


def _mm_mid(acc, den, res, b2d, w, aab, rep, relu):
    grid = N // BN
    if res is None:
        def body2(a, d, b, w_, ab, rp, g, h, asad, gm):
            _mm_mid_body(a, d, None, b, w_, ab, rp, g, h, asad, gm, relu=relu)
        return pl.pallas_call(
            body2,
            grid=(grid,),
            in_specs=[_row_spec(D), _row_spec(8), _full_spec(1, D),
                      _full_spec(D, D), _full_spec(D, 8), _full_spec(8, D)],
            out_specs=[_row_spec(D), _row_spec(D), _row_spec(8),
                       _full_spec(1, 8)],
            out_shape=[
                jax.ShapeDtypeStruct((N, D), jnp.float32),
                jax.ShapeDtypeStruct((N, D), jnp.float32),
                jax.ShapeDtypeStruct((N, 8), jnp.float32),
                jax.ShapeDtypeStruct((1, 8), jnp.float32),
            ],
        )(acc, den, b2d, w, aab, rep)
    body = partial(_mm_mid_body, relu=relu)
    return pl.pallas_call(
        body,
        grid=(grid,),
        in_specs=[_row_spec(D), _row_spec(8), _row_spec(D), _full_spec(1, D),
                  _full_spec(D, D), _full_spec(D, 8), _full_spec(8, D)],
        out_specs=[_row_spec(D), _row_spec(D), _row_spec(8), _full_spec(1, 8)],
        out_shape=[
            jax.ShapeDtypeStruct((N, D), jnp.float32),
            jax.ShapeDtypeStruct((N, D), jnp.float32),
            jax.ShapeDtypeStruct((N, 8), jnp.float32),
            jax.ShapeDtypeStruct((1, 8), jnp.float32),
        ],
    )(acc, den, res, b2d, w, aab, rep)


def _mm_final(acc, den, res, b2d, wf, bf2d, rep):
    grid = N // BN
    return pl.pallas_call(
        _mm_final_body,
        grid=(grid,),
        in_specs=[_row_spec(D), _row_spec(8), _row_spec(D), _full_spec(1, D),
                  _full_spec(D, OUT), _full_spec(1, OUT), _full_spec(8, D)],
        out_specs=[_row_spec(OUT), _row_spec(OUT)],
        out_shape=[
            jax.ShapeDtypeStruct((N, OUT), jnp.float32),
            jax.ShapeDtypeStruct((N, OUT), jnp.float32),
        ],
    )(acc, den, res, b2d, wf, bf2d, rep)


def _edge_phase(h, asad, gmax8, src, dst):
    as_ = asad[:, :4]
    ad_ = asad[:, 4:]
    gmax = gmax8[0, :4]
    t = as_[src] + ad_[dst]
    lr = jnp.maximum(t, 0.2 * t)
    gm = gmax[None, :] + ad_
    c = jnp.maximum(gm, 0.2 * gm)
    ex = jnp.exp(lr - c[dst])
    den = jax.ops.segment_sum(ex, dst, num_segments=N)
    msg = h[src] * jnp.repeat(ex, HID, axis=1)
    acc = jax.ops.segment_sum(msg, dst, num_segments=N)
    den8 = jnp.concatenate([den, den], axis=1)
    return acc, den8


def _make_aab(a_src, a_dst):
    z = jnp.zeros((HEADS, HID, 8), jnp.float32)
    hd = jnp.arange(HEADS)
    z = z.at[hd, :, hd].set(a_src[0])
    z = z.at[hd, :, hd + 4].set(a_dst[0])
    return z.reshape(D, 8)


def kernel(x, edge_index, W0, a_src0, a_dst0, b0, W1, a_src1, a_dst1, b1,
           W2, a_src2, a_dst2, b2, Wf, bf):
    src = edge_index[0]
    dst = edge_index[1]

    x_pad = jnp.pad(x, ((0, 0), (0, KPAD - IN)))
    w0_pad = jnp.pad(W0, ((0, KPAD - IN), (0, 0)))
    aab0 = _make_aab(a_src0, a_dst0)
    aab1 = _make_aab(a_src1, a_dst1)
    aab2 = _make_aab(a_src2, a_dst2)
    rep = jnp.repeat(jnp.eye(8, dtype=jnp.float32)[:4], HID, axis=1)
    b0_2 = b0[None, :]
    b1_2 = b1[None, :]
    b2_2 = b2[None, :]
    bf_2 = bf[None, :]

    h0, asad0, gmax0 = _mm_first(x_pad, w0_pad, aab0)
    acc0, den0 = _edge_phase(h0, asad0, gmax0, src, dst)
    g0, h1, asad1, gmax1 = _mm_mid(acc0, den0, None, b0_2, W1, aab1, rep,
                                   relu=False)
    acc1, den1 = _edge_phase(h1, asad1, gmax1, src, dst)
    g1, h2, asad2, gmax2 = _mm_mid(acc1, den1, g0, b1_2, W2, aab2, rep,
                                   relu=True)
    acc2, den2 = _edge_phase(h2, asad2, gmax2, src, dst)
    probs, logits = _mm_final(acc2, den2, g1, b2_2, Wf, bf_2, rep)
    return (probs, logits)


# speedup vs baseline: 5.6865x; 5.6865x over previous
from functools import partial

import jax
import jax.numpy as jnp
from jax.experimental import pallas as pl

N = 10000
E = 160000
IN = 131
HID = 256
HEADS = 4
OUT = 6
D = HID * HEADS
BN = 1000
KPAD = 256

_NEG = -3.0e38


def _mm_first_body(x_ref, w_ref, aab_ref, h_ref, asad_ref, gmax_ref):
    i = pl.program_id(0)
    x = x_ref[...]
    h = jnp.dot(x, w_ref[...], preferred_element_type=jnp.float32)
    h_ref[...] = h
    asad = jnp.dot(h, aab_ref[...], preferred_element_type=jnp.float32)
    asad_ref[...] = asad
    part = jnp.max(asad, axis=0, keepdims=True)

    @pl.when(i == 0)
    def _():
        gmax_ref[...] = jnp.full((1, 8), _NEG, jnp.float32)

    gmax_ref[...] = jnp.maximum(gmax_ref[...], part)


def _mm_mid_body(acc_ref, den_ref, res_ref, b_ref, w_ref, aab_ref, rep_ref,
                 g_ref, h_ref, asad_ref, gmax_ref, *, relu):
    i = pl.program_id(0)
    den = jnp.dot(den_ref[...], rep_ref[...],
                  preferred_element_type=jnp.float32)
    g = acc_ref[...] / (den + 1e-16) + b_ref[...]
    if res_ref is not None:
        g = g + res_ref[...]
    if relu:
        g = jnp.maximum(g, 0.0)
    g_ref[...] = g
    h = jnp.dot(g, w_ref[...], preferred_element_type=jnp.float32)
    h_ref[...] = h
    asad = jnp.dot(h, aab_ref[...], preferred_element_type=jnp.float32)
    asad_ref[...] = asad
    part = jnp.max(asad, axis=0, keepdims=True)

    @pl.when(i == 0)
    def _():
        gmax_ref[...] = jnp.full((1, 8), _NEG, jnp.float32)

    gmax_ref[...] = jnp.maximum(gmax_ref[...], part)


def _mm_final_body(acc_ref, den_ref, res_ref, b_ref, wf_ref, bf_ref, rep_ref,
                   probs_ref, logits_ref):
    den = jnp.dot(den_ref[...], rep_ref[...],
                  preferred_element_type=jnp.float32)
    g = acc_ref[...] / (den + 1e-16) + b_ref[...] + res_ref[...]
    g = jnp.maximum(g, 0.0)
    logits = jnp.dot(g, wf_ref[...], preferred_element_type=jnp.float32)
    logits = logits + bf_ref[...]
    logits_ref[...] = logits
    z = logits - jnp.max(logits, axis=1, keepdims=True)
    ez = jnp.exp(z)
    probs_ref[...] = ez / jnp.sum(ez, axis=1, keepdims=True)


def _row_spec(cols):
    return pl.BlockSpec((BN, cols), lambda i: (i, 0))


def _full_spec(r, c):
    return pl.BlockSpec((r, c), lambda i: (0, 0))


def _mm_first(x_pad, w_pad, aab):
    grid = N // BN
    return pl.pallas_call(
        _mm_first_body,
        grid=(grid,),
        in_specs=[_row_spec(KPAD), _full_spec(KPAD, D), _full_spec(D, 8)],
        out_specs=[_row_spec(D), _row_spec(8), _full_spec(1, 8)],
        out_shape=[
            jax.ShapeDtypeStruct((N, D), jnp.float32),
            jax.ShapeDtypeStruct((N, 8), jnp.float32),
            jax.ShapeDtypeStruct((1, 8), jnp.float32),
        ],
    )(x_pad, w_pad, aab)


def _mm_mid(acc, den, res, b2d, w, aab, rep, relu):
    grid = N // BN
    if res is None:
        def body2(a, d, b, w_, ab, rp, g, h, asad, gm):
            _mm_mid_body(a, d, None, b, w_, ab, rp, g, h, asad, gm, relu=relu)
        return pl.pallas_call(
            body2,
            grid=(grid,),
            in_specs=[_row_spec(D), _row_spec(8), _full_spec(1, D),
                      _full_spec(D, D), _full_spec(D, 8), _full_spec(8, D)],
            out_specs=[_row_spec(D), _row_spec(D), _row_spec(8),
                       _full_spec(1, 8)],
            out_shape=[
                jax.ShapeDtypeStruct((N, D), jnp.float32),
                jax.ShapeDtypeStruct((N, D), jnp.float32),
                jax.ShapeDtypeStruct((N, 8), jnp.float32),
                jax.ShapeDtypeStruct((1, 8), jnp.float32),
            ],
        )(acc, den, b2d, w, aab, rep)
    body = partial(_mm_mid_body, relu=relu)
    return pl.pallas_call(
        body,
        grid=(grid,),
        in_specs=[_row_spec(D), _row_spec(8), _row_spec(D), _full_spec(1, D),
                  _full_spec(D, D), _full_spec(D, 8), _full_spec(8, D)],
        out_specs=[_row_spec(D), _row_spec(D), _row_spec(8), _full_spec(1, 8)],
        out_shape=[
            jax.ShapeDtypeStruct((N, D), jnp.float32),
            jax.ShapeDtypeStruct((N, D), jnp.float32),
            jax.ShapeDtypeStruct((N, 8), jnp.float32),
            jax.ShapeDtypeStruct((1, 8), jnp.float32),
        ],
    )(acc, den, res, b2d, w, aab, rep)


def _mm_final(acc, den, res, b2d, wf, bf2d, rep):
    grid = N // BN
    return pl.pallas_call(
        _mm_final_body,
        grid=(grid,),
        in_specs=[_row_spec(D), _row_spec(8), _row_spec(D), _full_spec(1, D),
                  _full_spec(D, OUT), _full_spec(1, OUT), _full_spec(8, D)],
        out_specs=[_row_spec(OUT), _row_spec(OUT)],
        out_shape=[
            jax.ShapeDtypeStruct((N, OUT), jnp.float32),
            jax.ShapeDtypeStruct((N, OUT), jnp.float32),
        ],
    )(acc, den, res, b2d, wf, bf2d, rep)


def _edge_phase(h, asad, gmax8, src, dst):
    as_ = asad[:, :4]
    ad_ = asad[:, 4:]
    gmax = gmax8[0, :4]
    t = as_[src] + ad_[dst]
    lr = jnp.maximum(t, 0.2 * t)
    gm = gmax[None, :] + ad_
    c = jnp.maximum(gm, 0.2 * gm)
    ex = jnp.exp(lr - c[dst])
    den = jax.ops.segment_sum(ex, dst, num_segments=N)
    msg = h[src] * jnp.repeat(ex, HID, axis=1)
    acc = jax.ops.segment_sum(msg, dst, num_segments=N)
    den8 = jnp.concatenate([den, den], axis=1)
    return acc, den8


def _make_aab(a_src, a_dst):
    z = jnp.zeros((HEADS, HID, 8), jnp.float32)
    hd = jnp.arange(HEADS)
    z = z.at[hd, :, hd].set(a_src[0])
    z = z.at[hd, :, hd + 4].set(a_dst[0])
    return z.reshape(D, 8)


def kernel(x, edge_index, W0, a_src0, a_dst0, b0, W1, a_src1, a_dst1, b1,
           W2, a_src2, a_dst2, b2, Wf, bf):
    src = edge_index[0]
    dst = edge_index[1]

    x_pad = jnp.pad(x, ((0, 0), (0, KPAD - IN)))
    w0_pad = jnp.pad(W0, ((0, KPAD - IN), (0, 0)))
    aab0 = _make_aab(a_src0, a_dst0)
    aab1 = _make_aab(a_src1, a_dst1)
    aab2 = _make_aab(a_src2, a_dst2)
    rep = jnp.repeat(jnp.eye(8, dtype=jnp.float32)[:, :4], HID, axis=1)
    b0_2 = b0[None, :]
    b1_2 = b1[None, :]
    b2_2 = b2[None, :]
    bf_2 = bf[None, :]

    h0, asad0, gmax0 = _mm_first(x_pad, w0_pad, aab0)
    acc0, den0 = _edge_phase(h0, asad0, gmax0, src, dst)
    g0, h1, asad1, gmax1 = _mm_mid(acc0, den0, None, b0_2, W1, aab1, rep,
                                   relu=False)
    acc1, den1 = _edge_phase(h1, asad1, gmax1, src, dst)
    g1, h2, asad2, gmax2 = _mm_mid(acc1, den1, g0, b1_2, W2, aab2, rep,
                                   relu=True)
    acc2, den2 = _edge_phase(h2, asad2, gmax2, src, dst)
    probs, logits = _mm_final(acc2, den2, g1, b2_2, Wf, bf_2, rep)
    return (probs, logits)


# speedup vs baseline: 11.9565x; 2.1026x over previous
from functools import partial

import jax
import jax.numpy as jnp
from jax import lax
from jax.experimental import pallas as pl
from jax.experimental.pallas import tpu as pltpu
from jax.experimental.pallas import tpu_sc as plsc

N = 10000
E = 160000
IN = 131
HID = 256
HEADS = 4
OUT = 6
D = HID * HEADS
BN = 1000
KPAD = 256

_NEG = -3.0e38


def _mm_first_body(x_ref, w_ref, aab_ref, h_ref, asad_ref, gmax_ref):
    i = pl.program_id(0)
    x = x_ref[...]
    h = jnp.dot(x, w_ref[...], preferred_element_type=jnp.float32)
    h_ref[...] = h
    asad = jnp.dot(h, aab_ref[...], preferred_element_type=jnp.float32)
    asad_ref[...] = asad
    part = jnp.max(asad, axis=0, keepdims=True)

    @pl.when(i == 0)
    def _():
        gmax_ref[...] = jnp.full((1, 8), _NEG, jnp.float32)

    gmax_ref[...] = jnp.maximum(gmax_ref[...], part)


def _mm_mid_body(acc_ref, den_ref, res_ref, b_ref, w_ref, aab_ref, rep_ref,
                 g_ref, h_ref, asad_ref, gmax_ref, *, relu):
    i = pl.program_id(0)
    den4 = jnp.sum(den_ref[...], axis=(0, 1))
    den = jnp.dot(den4, rep_ref[...],
                  preferred_element_type=jnp.float32)
    g = acc_ref[...] / (den + 1e-16) + b_ref[...]
    if res_ref is not None:
        g = g + res_ref[...]
    if relu:
        g = jnp.maximum(g, 0.0)
    g_ref[...] = g
    h = jnp.dot(g, w_ref[...], preferred_element_type=jnp.float32)
    h_ref[...] = h
    asad = jnp.dot(h, aab_ref[...], preferred_element_type=jnp.float32)
    asad_ref[...] = asad
    part = jnp.max(asad, axis=0, keepdims=True)

    @pl.when(i == 0)
    def _():
        gmax_ref[...] = jnp.full((1, 8), _NEG, jnp.float32)

    gmax_ref[...] = jnp.maximum(gmax_ref[...], part)


def _mm_final_body(acc_ref, den_ref, res_ref, b_ref, wf_ref, bf_ref, rep_ref,
                   probs_ref, logits_ref):
    den4 = jnp.sum(den_ref[...], axis=(0, 1))
    den = jnp.dot(den4, rep_ref[...],
                  preferred_element_type=jnp.float32)
    g = acc_ref[...] / (den + 1e-16) + b_ref[...] + res_ref[...]
    g = jnp.maximum(g, 0.0)
    logits = jnp.dot(g, wf_ref[...], preferred_element_type=jnp.float32)
    logits = logits + bf_ref[...]
    logits_ref[...] = logits
    z = logits - jnp.max(logits, axis=1, keepdims=True)
    ez = jnp.exp(z)
    probs_ref[...] = ez / jnp.sum(ez, axis=1, keepdims=True)


_DEN_SPEC = pl.BlockSpec((1, 16, 1000, 4), lambda i: (i, 0, 0, 0))


def _row_spec(cols):
    return pl.BlockSpec((BN, cols), lambda i: (i, 0))


def _full_spec(r, c):
    return pl.BlockSpec((r, c), lambda i: (0, 0))


def _mm_first(x_pad, w_pad, aab):
    grid = N // BN
    return pl.pallas_call(
        _mm_first_body,
        grid=(grid,),
        in_specs=[_row_spec(KPAD), _full_spec(KPAD, D), _full_spec(D, 8)],
        out_specs=[_row_spec(D), _row_spec(8), _full_spec(1, 8)],
        out_shape=[
            jax.ShapeDtypeStruct((N, D), jnp.float32),
            jax.ShapeDtypeStruct((N, 8), jnp.float32),
            jax.ShapeDtypeStruct((1, 8), jnp.float32),
        ],
    )(x_pad, w_pad, aab)


def _mm_mid(acc, den, res, b2d, w, aab, rep, relu):
    grid = N // BN
    if res is None:
        def body2(a, d, b, w_, ab, rp, g, h, asad, gm):
            _mm_mid_body(a, d, None, b, w_, ab, rp, g, h, asad, gm, relu=relu)
        return pl.pallas_call(
            body2,
            grid=(grid,),
            in_specs=[_row_spec(D), _DEN_SPEC, _full_spec(1, D),
                      _full_spec(D, D), _full_spec(D, 8), _full_spec(4, D)],
            out_specs=[_row_spec(D), _row_spec(D), _row_spec(8),
                       _full_spec(1, 8)],
            out_shape=[
                jax.ShapeDtypeStruct((N, D), jnp.float32),
                jax.ShapeDtypeStruct((N, D), jnp.float32),
                jax.ShapeDtypeStruct((N, 8), jnp.float32),
                jax.ShapeDtypeStruct((1, 8), jnp.float32),
            ],
        )(acc, den, b2d, w, aab, rep)
    body = partial(_mm_mid_body, relu=relu)
    return pl.pallas_call(
        body,
        grid=(grid,),
        in_specs=[_row_spec(D), _DEN_SPEC, _row_spec(D), _full_spec(1, D),
                  _full_spec(D, D), _full_spec(D, 8), _full_spec(4, D)],
        out_specs=[_row_spec(D), _row_spec(D), _row_spec(8), _full_spec(1, 8)],
        out_shape=[
            jax.ShapeDtypeStruct((N, D), jnp.float32),
            jax.ShapeDtypeStruct((N, D), jnp.float32),
            jax.ShapeDtypeStruct((N, 8), jnp.float32),
            jax.ShapeDtypeStruct((1, 8), jnp.float32),
        ],
    )(acc, den, res, b2d, w, aab, rep)


def _mm_final(acc, den, res, b2d, wf, bf2d, rep):
    grid = N // BN
    return pl.pallas_call(
        _mm_final_body,
        grid=(grid,),
        in_specs=[_row_spec(D), _DEN_SPEC, _row_spec(D), _full_spec(1, D),
                  _full_spec(D, OUT), _full_spec(1, OUT), _full_spec(4, D)],
        out_specs=[_row_spec(OUT), _row_spec(OUT)],
        out_shape=[
            jax.ShapeDtypeStruct((N, OUT), jnp.float32),
            jax.ShapeDtypeStruct((N, OUT), jnp.float32),
        ],
    )(acc, den, res, b2d, wf, bf2d, rep)


CH = 1000
NCHUNK = 10
CSTEPS = 5


def _sc_edge_body(h_hbm, ss_hbm, ds_hbm, asad_hbm, bnd_hbm, gmx_hbm,
                  acc_hbm, denp_hbm,
                  asb_v, adb_v, hbuf_v, denl_v, ssb_v, dsb_v, dlb_v,
                  bnd_v, gmx_v, acc_sh):
    cid = lax.axis_index("c")
    sid = lax.axis_index("s")
    pltpu.sync_copy(bnd_hbm, bnd_v)
    pltpu.sync_copy(gmx_hbm, gmx_v)
    zv = jnp.zeros((16,), jnp.float32)
    lane = lax.broadcasted_iota(jnp.int32, (16,), 0)
    rz_row = lane // 4
    rz_col = lane % 4

    for step in range(CSTEPS):
        chunk = 2 * step + cid
        base = chunk * CH
        for k in range(16):
            def zcol(j, _, k=k):
                hbuf_v[k, pl.ds(j * 16, 16)] = zv
                return 0
            lax.fori_loop(0, 64, zcol, 0)

        def zden(j, _):
            plsc.store_scatter(denl_v, [j * 4 + rz_row, rz_col], zv)
            return 0
        lax.fori_loop(0, CH // 4, zden, 0)

        @pl.when(sid < 15)
        def _():
            for r in range(4):
                pltpu.sync_copy(hbuf_v, acc_sh.at[pl.ds(sid * 64 + r * 16, 16)])

        @pl.when(sid == 15)
        def _():
            pltpu.sync_copy(hbuf_v, acc_sh.at[pl.ds(960, 16)])
            pltpu.sync_copy(hbuf_v, acc_sh.at[pl.ds(976, 16)])
            pltpu.sync_copy(hbuf_v.at[pl.ds(0, 8)], acc_sh.at[pl.ds(992, 8)])

        plsc.subcore_barrier()

        bvec = bnd_v[...]
        lo = jnp.sum(jnp.where(lane == chunk, bvec, 0))
        hi = jnp.sum(jnp.where(lane == chunk + 1, bvec, 0))
        ab0 = lo // 16
        ab1 = (hi + 15) // 16
        nbt = (ab1 - ab0 + 15) // 16
        bs = ab0 + sid * nbt
        be = jnp.minimum(bs + nbt, ab1)

        def batch(bi, _):
            eb = bi * 16
            pltpu.sync_copy(ss_hbm.at[pl.ds(eb, 16)], ssb_v)
            pltpu.sync_copy(ds_hbm.at[pl.ds(eb, 16)], dsb_v)
            sv = ssb_v[...]
            dv = dsb_v[...]
            ev = eb + lane
            msk = (ev >= lo) & (ev < hi)
            dl = jnp.clip(dv - base, 0, CH - 1)
            dlb_v[...] = dl
            pltpu.sync_copy(h_hbm.at[ssb_v], hbuf_v)
            pltpu.sync_copy(asad_hbm.at[ssb_v], asb_v)
            pltpu.sync_copy(asad_hbm.at[dsb_v], adb_v)
            gvec = gmx_v[...]
            exs = []
            for hd in range(HEADS):
                hdv = jnp.full((16,), hd, jnp.int32)
                av = plsc.load_gather(asb_v, [lane, hdv])
                bv = plsc.load_gather(adb_v, [lane, hdv + 4])
                t = av + bv
                lr = jnp.maximum(t, 0.2 * t)
                gb = gvec[hd] + bv
                c = jnp.maximum(gb, 0.2 * gb)
                ex = jnp.exp(lr - c)
                ex = jnp.where(msk, ex, 0.0)
                exs.append(ex)
                plsc.addupdate_scatter(denl_v, [dl, jnp.full((16,), hd,
                                                             jnp.int32)], ex)

            def srow(k, _):
                for hd in range(HEADS):
                    s = jnp.sum(jnp.where(lane == k, exs[hd], 0.0))
                    for j in range(16):
                        sl = pl.ds(hd * HID + j * 16, 16)
                        hbuf_v[k, sl] = hbuf_v[k, sl] * s
                return 0
            lax.fori_loop(0, 16, srow, 0)
            pltpu.sync_copy(hbuf_v, acc_sh.at[dlb_v], add=True)
            return 0
        lax.fori_loop(bs, be, batch, 0)

        pltpu.sync_copy(denl_v, denp_hbm.at[chunk, sid])
        plsc.subcore_barrier()

        @pl.when(sid < 15)
        def _():
            pltpu.sync_copy(acc_sh.at[pl.ds(sid * 64, 64)],
                            acc_hbm.at[pl.ds(base + sid * 64, 64)])

        @pl.when(sid == 15)
        def _():
            pltpu.sync_copy(acc_sh.at[pl.ds(960, 40)],
                            acc_hbm.at[pl.ds(base + 960, 40)])

        plsc.subcore_barrier()


def _sc_edge(h, ss, ds, asad, bnd16, gmx16):
    mesh = plsc.VectorSubcoreMesh(core_axis_name="c", subcore_axis_name="s")
    f = pl.kernel(
        _sc_edge_body,
        out_type=[
            jax.ShapeDtypeStruct((N, D), jnp.float32),
            jax.ShapeDtypeStruct((NCHUNK, 16, CH, 4), jnp.float32),
        ],
        mesh=mesh,
        compiler_params=pltpu.CompilerParams(needs_layout_passes=False,
                                             use_tc_tiling_on_sc=False),
        scratch_types=[
            pltpu.VMEM((16, 8), jnp.float32),
            pltpu.VMEM((16, 8), jnp.float32),
            pltpu.VMEM((16, D), jnp.float32),
            pltpu.VMEM((CH, 4), jnp.float32),
            pltpu.VMEM((16,), jnp.int32),
            pltpu.VMEM((16,), jnp.int32),
            pltpu.VMEM((16,), jnp.int32),
            pltpu.VMEM((16,), jnp.int32),
            pltpu.VMEM((16,), jnp.float32),
            pltpu.VMEM_SHARED((CH, D), jnp.float32),
        ],
    )
    return f(h, ss, ds, asad, bnd16, gmx16)


def _make_aab(a_src, a_dst):
    z = jnp.zeros((HEADS, HID, 8), jnp.float32)
    hd = jnp.arange(HEADS)
    z = z.at[hd, :, hd].set(a_src[0])
    z = z.at[hd, :, hd + 4].set(a_dst[0])
    return z.reshape(D, 8)


def kernel(x, edge_index, W0, a_src0, a_dst0, b0, W1, a_src1, a_dst1, b1,
           W2, a_src2, a_dst2, b2, Wf, bf):
    src = edge_index[0]
    dst = edge_index[1]

    x_pad = jnp.pad(x, ((0, 0), (0, KPAD - IN)))
    w0_pad = jnp.pad(W0, ((0, KPAD - IN), (0, 0)))
    aab0 = _make_aab(a_src0, a_dst0)
    aab1 = _make_aab(a_src1, a_dst1)
    aab2 = _make_aab(a_src2, a_dst2)
    rep = jnp.repeat(jnp.eye(4, dtype=jnp.float32), HID, axis=1)
    b0_2 = b0[None, :]
    b1_2 = b1[None, :]
    b2_2 = b2[None, :]
    bf_2 = bf[None, :]

    perm = jnp.argsort(dst)
    ss = src[perm]
    ds = dst[perm]
    bnd16 = jnp.zeros((16,), jnp.int32).at[:NCHUNK + 1].set(
        jnp.searchsorted(ds, jnp.arange(0, N + CH, CH, dtype=jnp.int32))
        .astype(jnp.int32))

    def edge(h, asad, gmax8):
        gmx16 = jnp.pad(gmax8[0], (0, 8))
        return _sc_edge(h, ss, ds, asad, bnd16, gmx16)

    h0, asad0, gmax0 = _mm_first(x_pad, w0_pad, aab0)
    acc0, den0 = edge(h0, asad0, gmax0)
    g0, h1, asad1, gmax1 = _mm_mid(acc0, den0, None, b0_2, W1, aab1, rep,
                                   relu=False)
    acc1, den1 = edge(h1, asad1, gmax1)
    g1, h2, asad2, gmax2 = _mm_mid(acc1, den1, g0, b1_2, W2, aab2, rep,
                                   relu=True)
    acc2, den2 = edge(h2, asad2, gmax2)
    probs, logits = _mm_final(acc2, den2, g1, b2_2, Wf, bf_2, rep)
    return (probs, logits)


# speedup vs baseline: 20.1319x; 1.6838x over previous
from functools import partial

import jax
import jax.numpy as jnp
from jax import lax
from jax.experimental import pallas as pl
from jax.experimental.pallas import tpu as pltpu
from jax.experimental.pallas import tpu_sc as plsc

N = 10000
E = 160000
IN = 131
HID = 256
HEADS = 4
OUT = 6
D = HID * HEADS
BN = 1000
KPAD = 256

_NEG = -3.0e38


def _mm_first_body(x_ref, w_ref, aab_ref, h_ref, asad_ref, gmax_ref):
    i = pl.program_id(0)
    x = x_ref[...]
    h = jnp.dot(x, w_ref[...], preferred_element_type=jnp.float32)
    h_ref[...] = h
    asad = jnp.dot(h, aab_ref[...], preferred_element_type=jnp.float32)
    asad_ref[...] = asad
    part = jnp.max(asad, axis=0, keepdims=True)

    @pl.when(i == 0)
    def _():
        gmax_ref[...] = jnp.full((1, 8), _NEG, jnp.float32)

    gmax_ref[...] = jnp.maximum(gmax_ref[...], part)


def _mm_mid_body(acc_ref, den_ref, res_ref, b_ref, w_ref, aab_ref, rep_ref,
                 g_ref, h_ref, asad_ref, gmax_ref, *, relu):
    i = pl.program_id(0)
    den4 = jnp.sum(den_ref[...], axis=(0, 1))
    den = jnp.dot(den4, rep_ref[...],
                  preferred_element_type=jnp.float32)
    g = acc_ref[...] / (den + 1e-16) + b_ref[...]
    if res_ref is not None:
        g = g + res_ref[...]
    if relu:
        g = jnp.maximum(g, 0.0)
    g_ref[...] = g
    h = jnp.dot(g, w_ref[...], preferred_element_type=jnp.float32)
    h_ref[...] = h
    asad = jnp.dot(h, aab_ref[...], preferred_element_type=jnp.float32)
    asad_ref[...] = asad
    part = jnp.max(asad, axis=0, keepdims=True)

    @pl.when(i == 0)
    def _():
        gmax_ref[...] = jnp.full((1, 8), _NEG, jnp.float32)

    gmax_ref[...] = jnp.maximum(gmax_ref[...], part)


def _mm_final_body(acc_ref, den_ref, res_ref, b_ref, wf_ref, bf_ref, rep_ref,
                   probs_ref, logits_ref):
    den4 = jnp.sum(den_ref[...], axis=(0, 1))
    den = jnp.dot(den4, rep_ref[...],
                  preferred_element_type=jnp.float32)
    g = acc_ref[...] / (den + 1e-16) + b_ref[...] + res_ref[...]
    g = jnp.maximum(g, 0.0)
    logits = jnp.dot(g, wf_ref[...], preferred_element_type=jnp.float32)
    logits = logits + bf_ref[...]
    logits_ref[...] = logits
    z = logits - jnp.max(logits, axis=1, keepdims=True)
    ez = jnp.exp(z)
    probs_ref[...] = ez / jnp.sum(ez, axis=1, keepdims=True)


_DEN_SPEC = pl.BlockSpec((1, 16, 1000, 4), lambda i: (i, 0, 0, 0))


def _row_spec(cols):
    return pl.BlockSpec((BN, cols), lambda i: (i, 0))


def _full_spec(r, c):
    return pl.BlockSpec((r, c), lambda i: (0, 0))


def _mm_first(x_pad, w_pad, aab):
    grid = N // BN
    return pl.pallas_call(
        _mm_first_body,
        grid=(grid,),
        in_specs=[_row_spec(KPAD), _full_spec(KPAD, D), _full_spec(D, 8)],
        out_specs=[_row_spec(D), _row_spec(8), _full_spec(1, 8)],
        out_shape=[
            jax.ShapeDtypeStruct((N, D), jnp.float32),
            jax.ShapeDtypeStruct((N, 8), jnp.float32),
            jax.ShapeDtypeStruct((1, 8), jnp.float32),
        ],
    )(x_pad, w_pad, aab)


def _mm_mid(acc, den, res, b2d, w, aab, rep, relu):
    grid = N // BN
    if res is None:
        def body2(a, d, b, w_, ab, rp, g, h, asad, gm):
            _mm_mid_body(a, d, None, b, w_, ab, rp, g, h, asad, gm, relu=relu)
        return pl.pallas_call(
            body2,
            grid=(grid,),
            in_specs=[_row_spec(D), _DEN_SPEC, _full_spec(1, D),
                      _full_spec(D, D), _full_spec(D, 8), _full_spec(4, D)],
            out_specs=[_row_spec(D), _row_spec(D), _row_spec(8),
                       _full_spec(1, 8)],
            out_shape=[
                jax.ShapeDtypeStruct((N, D), jnp.float32),
                jax.ShapeDtypeStruct((N, D), jnp.float32),
                jax.ShapeDtypeStruct((N, 8), jnp.float32),
                jax.ShapeDtypeStruct((1, 8), jnp.float32),
            ],
        )(acc, den, b2d, w, aab, rep)
    body = partial(_mm_mid_body, relu=relu)
    return pl.pallas_call(
        body,
        grid=(grid,),
        in_specs=[_row_spec(D), _DEN_SPEC, _row_spec(D), _full_spec(1, D),
                  _full_spec(D, D), _full_spec(D, 8), _full_spec(4, D)],
        out_specs=[_row_spec(D), _row_spec(D), _row_spec(8), _full_spec(1, 8)],
        out_shape=[
            jax.ShapeDtypeStruct((N, D), jnp.float32),
            jax.ShapeDtypeStruct((N, D), jnp.float32),
            jax.ShapeDtypeStruct((N, 8), jnp.float32),
            jax.ShapeDtypeStruct((1, 8), jnp.float32),
        ],
    )(acc, den, res, b2d, w, aab, rep)


def _mm_final(acc, den, res, b2d, wf, bf2d, rep):
    grid = N // BN
    return pl.pallas_call(
        _mm_final_body,
        grid=(grid,),
        in_specs=[_row_spec(D), _DEN_SPEC, _row_spec(D), _full_spec(1, D),
                  _full_spec(D, OUT), _full_spec(1, OUT), _full_spec(4, D)],
        out_specs=[_row_spec(OUT), _row_spec(OUT)],
        out_shape=[
            jax.ShapeDtypeStruct((N, OUT), jnp.float32),
            jax.ShapeDtypeStruct((N, OUT), jnp.float32),
        ],
    )(acc, den, res, b2d, wf, bf2d, rep)


CH = 1000
NCHUNK = 10
CSTEPS = 5


BLK = 240
EPAD = ((E + BLK - 1) // BLK) * BLK


def _sc_edge_body(h_hbm, ss_hbm, ds_hbm, asad_hbm, bnd_hbm, gmx_hbm,
                  acc_hbm, denp_hbm,
                  asb_v, adb_v, hb0, hb1, hb2, denl_v, ssb_v, dsb_v,
                  dlb0, dlb1, dlb2, exb_v, bnd_v, gmx_v, acc_sh,
                  sg0, sg1, sg2, sc0, sc1, sc2):
    cid = lax.axis_index("c")
    sid = lax.axis_index("s")
    pltpu.sync_copy(bnd_hbm, bnd_v)
    pltpu.sync_copy(gmx_hbm, gmx_v)
    zv = jnp.zeros((16,), jnp.float32)
    lane = lax.broadcasted_iota(jnp.int32, (16,), 0)
    rz_row = lane // 4
    rz_col = lane % 4
    hbs = (hb0, hb1, hb2)
    dlbs = (dlb0, dlb1, dlb2)
    sgs = (sg0, sg1, sg2)
    scs = (sc0, sc1, sc2)

    def _chunk(step, _):
        chunk = 2 * step + cid
        base = chunk * CH
        for k in range(16):
            def zcol(j, _, k=k):
                hb0[k, pl.ds(j * 16, 16)] = zv
                return 0
            lax.fori_loop(0, 64, zcol, 0)

        def zden(j, _):
            plsc.store_scatter(denl_v, [j * 4 + rz_row, rz_col], zv)
            return 0
        lax.fori_loop(0, CH // 4, zden, 0)

        @pl.when(sid < 15)
        def _():
            for r in range(4):
                pltpu.sync_copy(hb0, acc_sh.at[pl.ds(sid * 64 + r * 16, 16)])

        @pl.when(sid == 15)
        def _():
            pltpu.sync_copy(hb0, acc_sh.at[pl.ds(960, 16)])
            pltpu.sync_copy(hb0, acc_sh.at[pl.ds(976, 16)])
            pltpu.sync_copy(hb0.at[pl.ds(0, 8)], acc_sh.at[pl.ds(992, 8)])

        plsc.subcore_barrier()

        bvec = bnd_v[...]
        gvec = gmx_v[...]
        lo = jnp.sum(jnp.where(lane == chunk, bvec, 0))
        hi = jnp.sum(jnp.where(lane == chunk + 1, bvec, 0))
        a0 = lo // BLK
        a1 = (hi + BLK - 1) // BLK
        tb = (a1 - a0 + 15) // 16
        bs = a0 + sid * tb
        be = jnp.minimum(bs + tb, a1)

        def block(blk, _):
            e0 = blk * BLK
            pltpu.sync_copy(ss_hbm.at[pl.ds(e0, BLK)], ssb_v)
            pltpu.sync_copy(ds_hbm.at[pl.ds(e0, BLK)], dsb_v)
            for half, hn in ((0, 120), (1, 120)):
                hsl = pl.ds(half * 120, 120)
                pltpu.sync_copy(asad_hbm.at[ssb_v.at[hsl]], asb_v.at[hsl])
                pltpu.sync_copy(asad_hbm.at[dsb_v.at[hsl]], adb_v.at[hsl])

            def exf(sb, _):
                msk = ((e0 + sb * 16 + lane >= lo)
                       & (e0 + sb * 16 + lane < hi))
                for hd in range(HEADS):
                    hdv = jnp.full((16,), hd, jnp.int32)
                    row = sb * 16 + lane
                    av = plsc.load_gather(asb_v, [row, hdv])
                    bv = plsc.load_gather(adb_v, [row, hdv + 4])
                    t = av + bv
                    lr = jnp.maximum(t, 0.2 * t)
                    gb = gvec[hd] + bv
                    c = jnp.maximum(gb, 0.2 * gb)
                    ex = jnp.exp(lr - c)
                    exb_v[pl.ds((sb * 4 + hd) * 16, 16)] = (
                        jnp.where(msk, ex, 0.0))
                return 0
            lax.fori_loop(0, 15, exf, 0)

            pltpu.async_copy(h_hbm.at[ssb_v.at[pl.ds(0, 16)]], hb0, sg0)
            pltpu.async_copy(h_hbm.at[ssb_v.at[pl.ds(16, 16)]], hb1, sg1)

            def pipe(G, _):
                for b in range(3):
                    hb, dlb, sg, sc = hbs[b], dlbs[b], sgs[b], scs[b]
                    nb = (b + 2) % 3
                    g = G * 3 + b
                    pltpu.make_async_copy(
                        h_hbm.at[pl.ds(0, 16)], hb, sg).wait()

                    @pl.when(g >= 1)
                    def _(nb=nb):
                        pltpu.make_async_copy(
                            h_hbm.at[pl.ds(0, 16)], hbs[nb], scs[nb]).wait()

                    @pl.when(g <= 12)
                    def _(nb=nb, g=g):
                        pltpu.async_copy(
                            h_hbm.at[ssb_v.at[pl.ds((g + 2) * 16, 16)]],
                            hbs[nb], sgs[nb])

                    dv = plsc.load_gather(dsb_v, [g * 16 + lane])
                    dl = jnp.clip(dv - base, 0, CH - 1)
                    dlb[...] = dl
                    exvs = []
                    for hd in range(HEADS):
                        exv = plsc.load_gather(
                            exb_v, [(g * 4 + hd) * 16 + lane])
                        exvs.append(exv)
                        plsc.addupdate_scatter(
                            denl_v, [dl, jnp.full((16,), hd, jnp.int32)], exv)

                    def srow(k, _, hb=hb, exvs=exvs):
                        for hd in range(HEADS):
                            sca = jnp.sum(jnp.where(lane == k, exvs[hd], 0.0))
                            for j in range(16):
                                sl = pl.ds(hd * HID + j * 16, 16)
                                hb[k, sl] = hb[k, sl] * sca
                        return 0
                    lax.fori_loop(0, 16, srow, 0)
                    pltpu.async_copy(hb, acc_sh.at[dlb], sc, add=True)
                return 0
            lax.fori_loop(0, 5, pipe, 0)
            pltpu.make_async_copy(h_hbm.at[pl.ds(0, 16)], hb2, sc2).wait()
            return 0
        lax.fori_loop(bs, be, block, 0)

        pltpu.sync_copy(denl_v, denp_hbm.at[chunk, sid])
        plsc.subcore_barrier()

        @pl.when(sid < 15)
        def _():
            pltpu.sync_copy(acc_sh.at[pl.ds(sid * 64, 64)],
                            acc_hbm.at[pl.ds(base + sid * 64, 64)])

        @pl.when(sid == 15)
        def _():
            pltpu.sync_copy(acc_sh.at[pl.ds(960, 40)],
                            acc_hbm.at[pl.ds(base + 960, 40)])

        plsc.subcore_barrier()
        return 0

    lax.fori_loop(0, CSTEPS, _chunk, 0)


def _sc_edge(h, ss, ds, asad, bnd16, gmx16):
    mesh = plsc.VectorSubcoreMesh(core_axis_name="c", subcore_axis_name="s")
    f = pl.kernel(
        _sc_edge_body,
        out_type=[
            jax.ShapeDtypeStruct((N, D), jnp.float32),
            jax.ShapeDtypeStruct((NCHUNK, 16, CH, 4), jnp.float32),
        ],
        mesh=mesh,
        compiler_params=pltpu.CompilerParams(needs_layout_passes=False,
                                             use_tc_tiling_on_sc=False),
        scratch_types=[
            pltpu.VMEM((BLK, 8), jnp.float32),
            pltpu.VMEM((BLK, 8), jnp.float32),
            pltpu.VMEM((16, D), jnp.float32),
            pltpu.VMEM((16, D), jnp.float32),
            pltpu.VMEM((16, D), jnp.float32),
            pltpu.VMEM((CH, 4), jnp.float32),
            pltpu.VMEM((BLK,), jnp.int32),
            pltpu.VMEM((BLK,), jnp.int32),
            pltpu.VMEM((16,), jnp.int32),
            pltpu.VMEM((16,), jnp.int32),
            pltpu.VMEM((16,), jnp.int32),
            pltpu.VMEM((BLK * 4,), jnp.float32),
            pltpu.VMEM((16,), jnp.int32),
            pltpu.VMEM((16,), jnp.float32),
            pltpu.VMEM_SHARED((CH, D), jnp.float32),
            pltpu.SemaphoreType.DMA,
            pltpu.SemaphoreType.DMA,
            pltpu.SemaphoreType.DMA,
            pltpu.SemaphoreType.DMA,
            pltpu.SemaphoreType.DMA,
            pltpu.SemaphoreType.DMA,
        ],
    )
    return f(h, ss, ds, asad, bnd16, gmx16)


def _make_aab(a_src, a_dst):
    z = jnp.zeros((HEADS, HID, 8), jnp.float32)
    hd = jnp.arange(HEADS)
    z = z.at[hd, :, hd].set(a_src[0])
    z = z.at[hd, :, hd + 4].set(a_dst[0])
    return z.reshape(D, 8)


def kernel(x, edge_index, W0, a_src0, a_dst0, b0, W1, a_src1, a_dst1, b1,
           W2, a_src2, a_dst2, b2, Wf, bf):
    src = edge_index[0]
    dst = edge_index[1]

    x_pad = jnp.pad(x, ((0, 0), (0, KPAD - IN)))
    w0_pad = jnp.pad(W0, ((0, KPAD - IN), (0, 0)))
    aab0 = _make_aab(a_src0, a_dst0)
    aab1 = _make_aab(a_src1, a_dst1)
    aab2 = _make_aab(a_src2, a_dst2)
    rep = jnp.repeat(jnp.eye(4, dtype=jnp.float32), HID, axis=1)
    b0_2 = b0[None, :]
    b1_2 = b1[None, :]
    b2_2 = b2[None, :]
    bf_2 = bf[None, :]

    perm = jnp.argsort(dst)
    dss = dst[perm]
    bnd16 = jnp.zeros((16,), jnp.int32).at[:NCHUNK + 1].set(
        jnp.searchsorted(dss, jnp.arange(0, N + CH, CH, dtype=jnp.int32))
        .astype(jnp.int32))
    ss = jnp.pad(src[perm], (0, EPAD - E))
    ds = jnp.pad(dss, (0, EPAD - E))

    def edge(h, asad, gmax8):
        gmx16 = jnp.pad(gmax8[0], (0, 8))
        return _sc_edge(h, ss, ds, asad, bnd16, gmx16)

    h0, asad0, gmax0 = _mm_first(x_pad, w0_pad, aab0)
    acc0, den0 = edge(h0, asad0, gmax0)
    g0, h1, asad1, gmax1 = _mm_mid(acc0, den0, None, b0_2, W1, aab1, rep,
                                   relu=False)
    acc1, den1 = edge(h1, asad1, gmax1)
    g1, h2, asad2, gmax2 = _mm_mid(acc1, den1, g0, b1_2, W2, aab2, rep,
                                   relu=True)
    acc2, den2 = edge(h2, asad2, gmax2)
    probs, logits = _mm_final(acc2, den2, g1, b2_2, Wf, bf_2, rep)
    return (probs, logits)


# speedup vs baseline: 25.3287x; 1.2581x over previous
from functools import partial

import jax
import jax.numpy as jnp
from jax import lax
from jax.experimental import pallas as pl
from jax.experimental.pallas import tpu as pltpu
from jax.experimental.pallas import tpu_sc as plsc

N = 10000
E = 160000
IN = 131
HID = 256
HEADS = 4
OUT = 6
D = HID * HEADS
BN = 1000
KPAD = 256

_NEG = -3.0e38


def _mm_first_body(x_ref, w_ref, aab_ref, h_ref, asad_ref, gmax_ref):
    i = pl.program_id(0)
    x = x_ref[...]
    h = jnp.dot(x, w_ref[...], preferred_element_type=jnp.float32)
    h_ref[...] = h
    asad = jnp.dot(h, aab_ref[...], preferred_element_type=jnp.float32)
    asad_ref[...] = asad
    part = jnp.max(asad, axis=0, keepdims=True)

    @pl.when(i == 0)
    def _():
        gmax_ref[...] = jnp.full((1, 8), _NEG, jnp.float32)

    gmax_ref[...] = jnp.maximum(gmax_ref[...], part)


def _mm_mid_body(acc_ref, den_ref, res_ref, b_ref, w_ref, aab_ref, rep_ref,
                 g_ref, h_ref, asad_ref, gmax_ref, *, relu):
    i = pl.program_id(0)
    den4 = jnp.sum(den_ref[...], axis=(0, 1))
    den = jnp.dot(den4, rep_ref[...],
                  preferred_element_type=jnp.float32)
    g = acc_ref[...] / (den + 1e-16) + b_ref[...]
    if res_ref is not None:
        g = g + res_ref[...]
    if relu:
        g = jnp.maximum(g, 0.0)
    g_ref[...] = g
    h = jnp.dot(g, w_ref[...], preferred_element_type=jnp.float32)
    h_ref[...] = h
    asad = jnp.dot(h, aab_ref[...], preferred_element_type=jnp.float32)
    asad_ref[...] = asad
    part = jnp.max(asad, axis=0, keepdims=True)

    @pl.when(i == 0)
    def _():
        gmax_ref[...] = jnp.full((1, 8), _NEG, jnp.float32)

    gmax_ref[...] = jnp.maximum(gmax_ref[...], part)


def _mm_final_body(acc_ref, den_ref, res_ref, b_ref, wf_ref, bf_ref, rep_ref,
                   probs_ref, logits_ref):
    den4 = jnp.sum(den_ref[...], axis=(0, 1))
    den = jnp.dot(den4, rep_ref[...],
                  preferred_element_type=jnp.float32)
    g = acc_ref[...] / (den + 1e-16) + b_ref[...] + res_ref[...]
    g = jnp.maximum(g, 0.0)
    logits = jnp.dot(g, wf_ref[...], preferred_element_type=jnp.float32)
    logits = logits + bf_ref[...]
    logits_ref[...] = logits
    z = logits - jnp.max(logits, axis=1, keepdims=True)
    ez = jnp.exp(z)
    probs_ref[...] = ez / jnp.sum(ez, axis=1, keepdims=True)


_DEN_SPEC = pl.BlockSpec((1, 16, 1000, 4), lambda i: (i, 0, 0, 0))


def _row_spec(cols):
    return pl.BlockSpec((BN, cols), lambda i: (i, 0))


def _full_spec(r, c):
    return pl.BlockSpec((r, c), lambda i: (0, 0))


def _mm_first(x_pad, w_pad, aab):
    grid = N // BN
    return pl.pallas_call(
        _mm_first_body,
        grid=(grid,),
        in_specs=[_row_spec(KPAD), _full_spec(KPAD, D), _full_spec(D, 8)],
        out_specs=[_row_spec(D), _row_spec(8), _full_spec(1, 8)],
        out_shape=[
            jax.ShapeDtypeStruct((N, D), jnp.float32),
            jax.ShapeDtypeStruct((N, 8), jnp.float32),
            jax.ShapeDtypeStruct((1, 8), jnp.float32),
        ],
    )(x_pad, w_pad, aab)


def _mm_mid(acc, den, res, b2d, w, aab, rep, relu):
    grid = N // BN
    if res is None:
        def body2(a, d, b, w_, ab, rp, g, h, asad, gm):
            _mm_mid_body(a, d, None, b, w_, ab, rp, g, h, asad, gm, relu=relu)
        return pl.pallas_call(
            body2,
            grid=(grid,),
            in_specs=[_row_spec(D), _DEN_SPEC, _full_spec(1, D),
                      _full_spec(D, D), _full_spec(D, 8), _full_spec(4, D)],
            out_specs=[_row_spec(D), _row_spec(D), _row_spec(8),
                       _full_spec(1, 8)],
            out_shape=[
                jax.ShapeDtypeStruct((N, D), jnp.float32),
                jax.ShapeDtypeStruct((N, D), jnp.float32),
                jax.ShapeDtypeStruct((N, 8), jnp.float32),
                jax.ShapeDtypeStruct((1, 8), jnp.float32),
            ],
        )(acc, den, b2d, w, aab, rep)
    body = partial(_mm_mid_body, relu=relu)
    return pl.pallas_call(
        body,
        grid=(grid,),
        in_specs=[_row_spec(D), _DEN_SPEC, _row_spec(D), _full_spec(1, D),
                  _full_spec(D, D), _full_spec(D, 8), _full_spec(4, D)],
        out_specs=[_row_spec(D), _row_spec(D), _row_spec(8), _full_spec(1, 8)],
        out_shape=[
            jax.ShapeDtypeStruct((N, D), jnp.float32),
            jax.ShapeDtypeStruct((N, D), jnp.float32),
            jax.ShapeDtypeStruct((N, 8), jnp.float32),
            jax.ShapeDtypeStruct((1, 8), jnp.float32),
        ],
    )(acc, den, res, b2d, w, aab, rep)


def _mm_final(acc, den, res, b2d, wf, bf2d, rep):
    grid = N // BN
    return pl.pallas_call(
        _mm_final_body,
        grid=(grid,),
        in_specs=[_row_spec(D), _DEN_SPEC, _row_spec(D), _full_spec(1, D),
                  _full_spec(D, OUT), _full_spec(1, OUT), _full_spec(4, D)],
        out_specs=[_row_spec(OUT), _row_spec(OUT)],
        out_shape=[
            jax.ShapeDtypeStruct((N, OUT), jnp.float32),
            jax.ShapeDtypeStruct((N, OUT), jnp.float32),
        ],
    )(acc, den, res, b2d, wf, bf2d, rep)


CH = 1000
NCHUNK = 10
CSTEPS = 5


BLK = 240
EPAD = E + BLK


def _sc_edge_body(h_hbm, ss_hbm, ds_hbm, asad_hbm, bnd_hbm, gmx_hbm,
                  acc_hbm, denp_hbm,
                  asb_v, adb_v, hb0, hb1, hb2, denl_v, ssb_v, dsb_v,
                  dlb0, dlb1, dlb2, bnd_v, gmx_v, acc_sh,
                  sg0, sg1, sg2, sc0, sc1, sc2, stgA, stgB):
    cid = lax.axis_index("c")
    sid = lax.axis_index("s")
    pltpu.sync_copy(bnd_hbm, bnd_v)
    pltpu.sync_copy(gmx_hbm, gmx_v)
    zv = jnp.zeros((16,), jnp.float32)
    lane = lax.broadcasted_iota(jnp.int32, (16,), 0)
    rz_row = lane // 4
    rz_col = lane % 4
    hbs = (hb0, hb1, hb2)
    dlbs = (dlb0, dlb1, dlb2)
    sgs = (sg0, sg1, sg2)
    scs = (sc0, sc1, sc2)

    def _stage(pofs, est, sem):
        pltpu.async_copy(ss_hbm.at[pl.ds(est, BLK)],
                         ssb_v.at[pl.ds(pofs, BLK)], sem)
        pltpu.async_copy(ds_hbm.at[pl.ds(est, BLK)],
                         dsb_v.at[pl.ds(pofs, BLK)], sem)

    def _stage_alpha(pofs, sem):
        for half in range(2):
            hsl = pl.ds(pofs + half * 120, 120)
            pltpu.async_copy(asad_hbm.at[ssb_v.at[hsl]], asb_v.at[hsl], sem)
            pltpu.async_copy(asad_hbm.at[dsb_v.at[hsl]], adb_v.at[hsl], sem)

    def _wait_stage(pofs, sem):
        pltpu.make_async_copy(ss_hbm.at[pl.ds(0, BLK)],
                              ssb_v.at[pl.ds(pofs, BLK)], sem).wait()
        pltpu.make_async_copy(ss_hbm.at[pl.ds(0, BLK)],
                              dsb_v.at[pl.ds(pofs, BLK)], sem).wait()

    def _wait_alpha(pofs, sem):
        for half in range(2):
            hsl = pl.ds(pofs + half * 120, 120)
            pltpu.make_async_copy(asad_hbm.at[pl.ds(0, 120)],
                                  asb_v.at[hsl], sem).wait()
            pltpu.make_async_copy(asad_hbm.at[pl.ds(0, 120)],
                                  adb_v.at[hsl], sem).wait()

    def _chunk(step, _):
        chunk = 2 * step + cid
        base = chunk * CH
        for k in range(16):
            def zcol(j, _, k=k):
                hb0[k, pl.ds(j * 16, 16)] = zv
                return 0
            lax.fori_loop(0, 64, zcol, 0)

        def zden(j, _):
            plsc.store_scatter(denl_v, [j * 4 + rz_row, rz_col], zv)
            return 0
        lax.fori_loop(0, CH // 4, zden, 0)

        @pl.when(sid < 15)
        def _():
            for r in range(4):
                pltpu.sync_copy(hb0, acc_sh.at[pl.ds(sid * 64 + r * 16, 16)])

        @pl.when(sid == 15)
        def _():
            pltpu.sync_copy(hb0, acc_sh.at[pl.ds(960, 16)])
            pltpu.sync_copy(hb0, acc_sh.at[pl.ds(976, 16)])
            pltpu.sync_copy(hb0.at[pl.ds(0, 8)], acc_sh.at[pl.ds(992, 8)])

        plsc.subcore_barrier()

        bvec = bnd_v[...]
        gvec = gmx_v[...]
        lo = jnp.sum(jnp.where(lane == chunk, bvec, 0))
        hi = jnp.sum(jnp.where(lane == chunk + 1, bvec, 0))
        sb0 = lo // 16
        nsb = (hi + 15) // 16 - sb0
        q = nsb // 16
        r = nsb % 16
        q0 = sb0 + sid * q + jnp.minimum(sid, r)
        cnt = q + (sid < r).astype(jnp.int32)
        ngr = (cnt + 2) // 3
        nwin = (cnt + 14) // 15

        @pl.when(cnt > 0)
        def _():
            _stage(0, q0 * 16, stgA)
            _wait_stage(0, stgA)
            _stage_alpha(0, stgB)
            _wait_alpha(0, stgB)
            pltpu.async_copy(h_hbm.at[ssb_v.at[pl.ds(0, 16)]], hb0, sg0)
            pltpu.async_copy(h_hbm.at[ssb_v.at[pl.ds(16, 16)]], hb1, sg1)

            def pipe(G, _):
                for b in range(3):
                    hb, dlb, sg, sc = hbs[b], dlbs[b], sgs[b], scs[b]
                    nb = (b + 2) % 3
                    g = G * 3 + b
                    wg = g // 15
                    gof = (wg % 2) * BLK + (g % 15) * 16
                    pltpu.make_async_copy(
                        h_hbm.at[pl.ds(0, 16)], hb, sg).wait()

                    @pl.when(g >= 1)
                    def _(nb=nb):
                        pltpu.make_async_copy(
                            h_hbm.at[pl.ds(0, 16)], hbs[nb], scs[nb]).wait()

                    nxt = wg + 1 < nwin
                    p1 = ((wg + 1) % 2) * BLK

                    @pl.when((g % 15 == 4) & nxt)
                    def _(p1=p1, wg=wg):
                        _stage(p1, (q0 + (wg + 1) * 15) * 16, stgA)

                    @pl.when((g % 15 == 6) & nxt)
                    def _(p1=p1):
                        _wait_stage(p1, stgA)
                        _stage_alpha(p1, stgB)

                    @pl.when((g % 15 == 13) & nxt)
                    def _(p1=p1):
                        _wait_alpha(p1, stgB)

                    @pl.when(g < ngr * 3 - 2)
                    def _(nb=nb, g=g):
                        o2 = (((g + 2) // 15) % 2) * BLK + ((g + 2) % 15) * 16
                        pltpu.async_copy(
                            h_hbm.at[ssb_v.at[pl.ds(o2, 16)]],
                            hbs[nb], sgs[nb])

                    dv = dsb_v[pl.ds(gof, 16)]
                    dl = jnp.clip(dv - base, 0, CH - 1)
                    dlb[...] = dl
                    ev = (q0 + g) * 16 + lane
                    msk = (ev >= lo) & (ev < hi) & (g < cnt)
                    exvs = []
                    for hd in range(HEADS):
                        hdv = jnp.full((16,), hd, jnp.int32)
                        av = plsc.load_gather(asb_v, [gof + lane, hdv])
                        bv = plsc.load_gather(adb_v, [gof + lane, hdv + 4])
                        t = av + bv
                        lr = jnp.maximum(t, 0.2 * t)
                        gb = gvec[hd] + bv
                        c = jnp.maximum(gb, 0.2 * gb)
                        exv = jnp.where(msk, jnp.exp(lr - c), 0.0)
                        exvs.append(exv)
                        plsc.addupdate_scatter(denl_v, [dl, hdv], exv)

                    def srow(k, _, hb=hb, exvs=exvs):
                        for hd in range(HEADS):
                            sca = jnp.sum(jnp.where(lane == k, exvs[hd], 0.0))
                            for j in range(16):
                                sl = pl.ds(hd * HID + j * 16, 16)
                                hb[k, sl] = hb[k, sl] * sca
                        return 0
                    lax.fori_loop(0, 16, srow, 0)
                    pltpu.async_copy(hb, acc_sh.at[dlb], sc, add=True)
                return 0
            lax.fori_loop(0, ngr, pipe, 0)
            pltpu.make_async_copy(h_hbm.at[pl.ds(0, 16)], hb2, sc2).wait()

        pltpu.sync_copy(denl_v, denp_hbm.at[chunk, sid])
        plsc.subcore_barrier()

        @pl.when(sid < 15)
        def _():
            pltpu.sync_copy(acc_sh.at[pl.ds(sid * 64, 64)],
                            acc_hbm.at[pl.ds(base + sid * 64, 64)])

        @pl.when(sid == 15)
        def _():
            pltpu.sync_copy(acc_sh.at[pl.ds(960, 40)],
                            acc_hbm.at[pl.ds(base + 960, 40)])

        plsc.subcore_barrier()
        return 0

    lax.fori_loop(0, CSTEPS, _chunk, 0)


def _sc_edge(h, ss, ds, asad, bnd16, gmx16):
    mesh = plsc.VectorSubcoreMesh(core_axis_name="c", subcore_axis_name="s")
    f = pl.kernel(
        _sc_edge_body,
        out_type=[
            jax.ShapeDtypeStruct((N, D), jnp.float32),
            jax.ShapeDtypeStruct((NCHUNK, 16, CH, 4), jnp.float32),
        ],
        mesh=mesh,
        compiler_params=pltpu.CompilerParams(needs_layout_passes=False,
                                             use_tc_tiling_on_sc=False),
        scratch_types=[
            pltpu.VMEM((2 * BLK, 8), jnp.float32),
            pltpu.VMEM((2 * BLK, 8), jnp.float32),
            pltpu.VMEM((16, D), jnp.float32),
            pltpu.VMEM((16, D), jnp.float32),
            pltpu.VMEM((16, D), jnp.float32),
            pltpu.VMEM((CH, 4), jnp.float32),
            pltpu.VMEM((2 * BLK,), jnp.int32),
            pltpu.VMEM((2 * BLK,), jnp.int32),
            pltpu.VMEM((16,), jnp.int32),
            pltpu.VMEM((16,), jnp.int32),
            pltpu.VMEM((16,), jnp.int32),
            pltpu.VMEM((16,), jnp.int32),
            pltpu.VMEM((16,), jnp.float32),
            pltpu.VMEM_SHARED((CH, D), jnp.float32),
            pltpu.SemaphoreType.DMA,
            pltpu.SemaphoreType.DMA,
            pltpu.SemaphoreType.DMA,
            pltpu.SemaphoreType.DMA,
            pltpu.SemaphoreType.DMA,
            pltpu.SemaphoreType.DMA,
            pltpu.SemaphoreType.DMA,
            pltpu.SemaphoreType.DMA,
        ],
    )
    return f(h, ss, ds, asad, bnd16, gmx16)


def _make_aab(a_src, a_dst):
    z = jnp.zeros((HEADS, HID, 8), jnp.float32)
    hd = jnp.arange(HEADS)
    z = z.at[hd, :, hd].set(a_src[0])
    z = z.at[hd, :, hd + 4].set(a_dst[0])
    return z.reshape(D, 8)


def kernel(x, edge_index, W0, a_src0, a_dst0, b0, W1, a_src1, a_dst1, b1,
           W2, a_src2, a_dst2, b2, Wf, bf):
    src = edge_index[0]
    dst = edge_index[1]

    x_pad = jnp.pad(x, ((0, 0), (0, KPAD - IN)))
    w0_pad = jnp.pad(W0, ((0, KPAD - IN), (0, 0)))
    aab0 = _make_aab(a_src0, a_dst0)
    aab1 = _make_aab(a_src1, a_dst1)
    aab2 = _make_aab(a_src2, a_dst2)
    rep = jnp.repeat(jnp.eye(4, dtype=jnp.float32), HID, axis=1)
    b0_2 = b0[None, :]
    b1_2 = b1[None, :]
    b2_2 = b2[None, :]
    bf_2 = bf[None, :]

    perm = jnp.argsort(dst)
    dss = dst[perm]
    bnd16 = jnp.zeros((16,), jnp.int32).at[:NCHUNK + 1].set(
        jnp.searchsorted(dss, jnp.arange(0, N + CH, CH, dtype=jnp.int32))
        .astype(jnp.int32))
    ss = jnp.pad(src[perm], (0, EPAD - E))
    ds = jnp.pad(dss, (0, EPAD - E))

    def edge(h, asad, gmax8):
        gmx16 = jnp.pad(gmax8[0], (0, 8))
        return _sc_edge(h, ss, ds, asad, bnd16, gmx16)

    h0, asad0, gmax0 = _mm_first(x_pad, w0_pad, aab0)
    acc0, den0 = edge(h0, asad0, gmax0)
    g0, h1, asad1, gmax1 = _mm_mid(acc0, den0, None, b0_2, W1, aab1, rep,
                                   relu=False)
    acc1, den1 = edge(h1, asad1, gmax1)
    g1, h2, asad2, gmax2 = _mm_mid(acc1, den1, g0, b1_2, W2, aab2, rep,
                                   relu=True)
    acc2, den2 = edge(h2, asad2, gmax2)
    probs, logits = _mm_final(acc2, den2, g1, b2_2, Wf, bf_2, rep)
    return (probs, logits)
